# Initial kernel scaffold; baseline (speedup 1.0000x reference)
import jax
import jax.numpy as jnp
from jax.experimental import pallas as pl


def kernel(x, W_qkv, W_o, W_iq, W_ik):
    raise NotImplementedError("write your pallas kernel here")


# speedup vs baseline: 11.3199x; 11.3199x over previous
import math

import jax
import jax.numpy as jnp
from jax.experimental import pallas as pl

D_MODEL = 2048
N_HEADS = 16
D_K = 128
IH = 4
ID = 64
IDF = IH * ID
TOPK = 512
S = 2048

BQ = 256
N_BLK = S // BQ
NEG = -3.0e38

DN_NN = (((1,), (0,)), ((), ()))
DN_NT = (((1,), (1,)), ((), ()))


def _split3(a):
    h = a.astype(jnp.bfloat16)
    r = a - h.astype(jnp.float32)
    m = r.astype(jnp.bfloat16)
    l = (r - m.astype(jnp.float32)).astype(jnp.bfloat16)
    return h, m, l


def _mm6(a, b, dn):
    ah, am, al = _split3(a)
    bh, bm, bl = _split3(b)
    f = jnp.float32
    acc = jax.lax.dot_general(al, bh, dn, preferred_element_type=f)
    acc += jax.lax.dot_general(am, bm, dn, preferred_element_type=f)
    acc += jax.lax.dot_general(ah, bl, dn, preferred_element_type=f)
    acc += jax.lax.dot_general(am, bh, dn, preferred_element_type=f)
    acc += jax.lax.dot_general(ah, bm, dn, preferred_element_type=f)
    acc += jax.lax.dot_general(ah, bh, dn, preferred_element_type=f)
    return acc


def _mm3(a, b, dn):
    ah = a.astype(jnp.bfloat16)
    al = (a - ah.astype(jnp.float32)).astype(jnp.bfloat16)
    bh = b.astype(jnp.bfloat16)
    bl = (b - bh.astype(jnp.float32)).astype(jnp.bfloat16)
    f = jnp.float32
    acc = jax.lax.dot_general(ah, bl, dn, preferred_element_type=f)
    acc += jax.lax.dot_general(al, bh, dn, preferred_element_type=f)
    acc += jax.lax.dot_general(ah, bh, dn, preferred_element_type=f)
    return acc


def _bf16_dot(a, b, dn):
    return jax.lax.dot_general(a.astype(jnp.bfloat16), b.astype(jnp.bfloat16),
                               dn, preferred_element_type=jnp.float32)


def _qk_kernel(x_ref, wq_ref, wk_ref, cos_ref, sin_ref, q_ref, k_ref):
    xb = x_ref[...]
    q = _bf16_dot(xb, wq_ref[...], DN_NN)
    k = _bf16_dot(xb, wk_ref[...], DN_NN)
    cos = cos_ref[...]
    sin = sin_ref[...]

    def rope(m):
        outs = []
        for h in range(N_HEADS):
            hm = m[:, h * D_K:(h + 1) * D_K]
            sw = jnp.concatenate([hm[:, D_K // 2:], hm[:, :D_K // 2]], axis=1)
            outs.append(hm * cos + sw * sin)
        return jnp.concatenate(outs, axis=1)

    q_ref[...] = rope(q)
    k_ref[...] = rope(k)


def _v_idx_kernel(x_ref, wv_ref, wiq_ref, wik_ref, v_ref, iq_ref, ik_ref):
    xb = x_ref[...]
    v_ref[...] = _bf16_dot(xb, wv_ref[...], DN_NN)
    iq_ref[...] = _bf16_dot(xb, wiq_ref[...], DN_NN)
    ik_ref[...] = _bf16_dot(xb, wik_ref[...], DN_NN)


def _select_kernel(iq_ref, ik_ref, bias_ref):
    qi = pl.program_id(0)
    row0 = qi * BQ
    scores = _bf16_dot(iq_ref[...], ik_ref[...], DN_NT) * (1.0 / math.sqrt(ID))
    rows = row0 + jax.lax.broadcasted_iota(jnp.int32, (BQ, S), 0)
    cols = jax.lax.broadcasted_iota(jnp.int32, (BQ, S), 1)
    causal_ok = cols <= rows
    sm = jnp.where(causal_ok, scores, NEG)

    bits = jax.lax.bitcast_convert_type(sm, jnp.int32)
    key = jnp.where(bits >= 0, bits, bits ^ jnp.int32(0x7FFFFFFF))
    prefix = jnp.full((BQ, 1), jnp.int32(-2**31))
    for bit in range(31, -1, -1):
        if bit == 31:
            cand = jnp.zeros_like(prefix)
        else:
            cand = prefix + jnp.int32(1 << bit)
        cnt = jnp.sum((key >= cand).astype(jnp.float32), axis=1, keepdims=True)
        prefix = jnp.where(cnt >= TOPK, cand, prefix)
    allowed = causal_ok & (key >= prefix)
    bias_ref[...] = jnp.where(allowed, 0.0, NEG)


def _attn_kernel(bias_ref, q_ref, k_ref, v_ref, out_ref):
    bias = bias_ref[...]
    qb = q_ref[...].astype(jnp.bfloat16)
    inv_sqrt_dk = 1.0 / math.sqrt(D_K)
    outs = []
    for h in range(N_HEADS):
        qh = qb[:, h * D_K:(h + 1) * D_K]
        kh = k_ref[:, h * D_K:(h + 1) * D_K].astype(jnp.bfloat16)
        logits = jax.lax.dot_general(
            qh, kh, DN_NT, preferred_element_type=jnp.float32) * inv_sqrt_dk
        logits = logits + bias
        m = jnp.max(logits, axis=1, keepdims=True)
        e = jnp.exp(logits - m)
        p = (e / jnp.sum(e, axis=1, keepdims=True)).astype(jnp.bfloat16)
        vh = v_ref[:, h * D_K:(h + 1) * D_K].astype(jnp.bfloat16)
        outs.append(jax.lax.dot_general(
            p, vh, DN_NN, preferred_element_type=jnp.float32))
    out_ref[...] = jnp.concatenate(outs, axis=1)


def _outproj_kernel(a_ref, wo_ref, out_ref):
    out_ref[...] = _bf16_dot(a_ref[...], wo_ref[...], DN_NN)


def kernel(x, W_qkv, W_o, W_iq, W_ik):
    b, s, dm = x.shape
    x2 = x[0]
    W = W_qkv.reshape(dm, 3, N_HEADS, D_K)
    perm = jnp.concatenate([jnp.arange(0, D_K, 2), jnp.arange(1, D_K, 2)])
    Wq = W[:, 0][:, :, perm].reshape(dm, D_MODEL)
    Wk = W[:, 1][:, :, perm].reshape(dm, D_MODEL)
    Wv = W[:, 2].reshape(dm, D_MODEL)

    pos = jnp.arange(s, dtype=jnp.float32)
    freqs = 1.0 / (10000.0 ** (jnp.arange(0, D_K, 2, dtype=jnp.float32) / D_K))
    ang = pos[:, None] * freqs[None, :]
    cosv = jnp.cos(ang)
    sinv = jnp.sin(ang)
    cos2 = jnp.concatenate([cosv, cosv], axis=1)
    sin2 = jnp.concatenate([-sinv, sinv], axis=1)

    blk = lambda shape: pl.BlockSpec(shape, lambda i: (i, 0))
    cst = lambda shape: pl.BlockSpec(shape, lambda i: (0, 0))
    f32 = jnp.float32

    q, k = pl.pallas_call(
        _qk_kernel,
        grid=(N_BLK,),
        in_specs=[blk((BQ, D_MODEL)), cst((D_MODEL, D_MODEL)),
                  cst((D_MODEL, D_MODEL)), blk((BQ, D_K)), blk((BQ, D_K))],
        out_specs=[blk((BQ, D_MODEL)), blk((BQ, D_MODEL))],
        out_shape=[jax.ShapeDtypeStruct((S, D_MODEL), f32)] * 2,
    )(x2, Wq, Wk, cos2, sin2)

    v, iq, ik = pl.pallas_call(
        _v_idx_kernel,
        grid=(N_BLK,),
        in_specs=[blk((BQ, D_MODEL)), cst((D_MODEL, D_MODEL)),
                  cst((D_MODEL, IDF)), cst((D_MODEL, IDF))],
        out_specs=[blk((BQ, D_MODEL)), blk((BQ, IDF)), blk((BQ, IDF))],
        out_shape=[jax.ShapeDtypeStruct((S, D_MODEL), f32),
                   jax.ShapeDtypeStruct((S, IDF), f32),
                   jax.ShapeDtypeStruct((S, IDF), f32)],
    )(x2, Wv, W_iq, W_ik)

    bias = pl.pallas_call(
        _select_kernel,
        grid=(N_BLK,),
        in_specs=[blk((BQ, IDF)), cst((S, IDF))],
        out_specs=blk((BQ, S)),
        out_shape=jax.ShapeDtypeStruct((S, S), f32),
    )(iq, ik)

    attnout = pl.pallas_call(
        _attn_kernel,
        grid=(N_BLK,),
        in_specs=[blk((BQ, S)), blk((BQ, D_MODEL)), cst((S, D_MODEL)),
                  cst((S, D_MODEL))],
        out_specs=blk((BQ, D_MODEL)),
        out_shape=jax.ShapeDtypeStruct((S, D_MODEL), f32),
    )(bias, q, k, v)

    out = pl.pallas_call(
        _outproj_kernel,
        grid=(N_BLK,),
        in_specs=[blk((BQ, D_MODEL)), cst((D_MODEL, D_MODEL))],
        out_specs=blk((BQ, D_MODEL)),
        out_shape=jax.ShapeDtypeStruct((S, D_MODEL), f32),
    )(attnout, W_o)

    return out[None, :, :]


# speedup vs baseline: 11.3442x; 1.0021x over previous
import math

import jax
import jax.numpy as jnp
from jax.experimental import pallas as pl

D_MODEL = 2048
N_HEADS = 16
D_K = 128
IH = 4
ID = 64
IDF = IH * ID
TOPK = 512
S = 2048

BQ = 256
N_BLK = S // BQ
NEG = -3.0e38

DN_NN = (((1,), (0,)), ((), ()))
DN_NT = (((1,), (1,)), ((), ()))


def _bf16_dot(a, b, dn):
    return jax.lax.dot_general(a.astype(jnp.bfloat16), b.astype(jnp.bfloat16),
                               dn, preferred_element_type=jnp.float32)


def _qk_kernel(x_ref, wq_ref, wk_ref, cos_ref, sin_ref, q_ref, k_ref):
    xb = x_ref[...]
    q = _bf16_dot(xb, wq_ref[...], DN_NN)
    k = _bf16_dot(xb, wk_ref[...], DN_NN)
    cos = cos_ref[...]
    sin = sin_ref[...]

    def rope(m):
        outs = []
        for h in range(N_HEADS):
            hm = m[:, h * D_K:(h + 1) * D_K]
            sw = jnp.concatenate([hm[:, D_K // 2:], hm[:, :D_K // 2]], axis=1)
            outs.append(hm * cos + sw * sin)
        return jnp.concatenate(outs, axis=1)

    q_ref[...] = rope(q)
    k_ref[...] = rope(k)


def _v_idx_kernel(x_ref, wv_ref, wiq_ref, wik_ref, v_ref, iq_ref, ik_ref):
    xb = x_ref[...]
    v_ref[...] = _bf16_dot(xb, wv_ref[...], DN_NN)
    iq_ref[...] = _bf16_dot(xb, wiq_ref[...], DN_NN)
    ik_ref[...] = _bf16_dot(xb, wik_ref[...], DN_NN)


def _select_kernel(iq_ref, ik_ref, bias_ref):
    qi = pl.program_id(0)
    row0 = qi * BQ
    scores = _bf16_dot(iq_ref[...], ik_ref[...], DN_NT) * (1.0 / math.sqrt(ID))
    rows = row0 + jax.lax.broadcasted_iota(jnp.int32, (BQ, S), 0)
    cols = jax.lax.broadcasted_iota(jnp.int32, (BQ, S), 1)
    causal_ok = cols <= rows
    sm = jnp.where(causal_ok, scores, NEG)

    bits = jax.lax.bitcast_convert_type(sm, jnp.int32)
    key = jnp.where(bits >= 0, bits, bits ^ jnp.int32(0x7FFFFFFF))
    prefix = jnp.full((BQ, 1), jnp.int32(-2**31))
    for bit in range(31, -1, -1):
        if bit == 31:
            cand = jnp.zeros_like(prefix)
        else:
            cand = prefix + jnp.int32(1 << bit)
        cnt = jnp.sum((key >= cand).astype(jnp.float32), axis=1, keepdims=True)
        prefix = jnp.where(cnt >= TOPK, cand, prefix)
    allowed = causal_ok & (key >= prefix)
    bias_ref[...] = jnp.where(allowed, 0.0, NEG)


def _attn_kernel(bias_ref, q_ref, k_ref, v_ref, out_ref):
    bias = bias_ref[...]
    qb = q_ref[...].astype(jnp.bfloat16)
    inv_sqrt_dk = 1.0 / math.sqrt(D_K)
    outs = []
    for h in range(N_HEADS):
        qh = qb[:, h * D_K:(h + 1) * D_K]
        kh = k_ref[:, h * D_K:(h + 1) * D_K].astype(jnp.bfloat16)
        logits = jax.lax.dot_general(
            qh, kh, DN_NT, preferred_element_type=jnp.float32) * inv_sqrt_dk
        logits = logits + bias
        m = jnp.max(logits, axis=1, keepdims=True)
        e = jnp.exp(logits - m)
        p = (e / jnp.sum(e, axis=1, keepdims=True)).astype(jnp.bfloat16)
        vh = v_ref[:, h * D_K:(h + 1) * D_K].astype(jnp.bfloat16)
        outs.append(jax.lax.dot_general(
            p, vh, DN_NN, preferred_element_type=jnp.float32))
    out_ref[...] = jnp.concatenate(outs, axis=1)


def _outproj_kernel(a_ref, wo_ref, out_ref):
    out_ref[...] = _bf16_dot(a_ref[...], wo_ref[...], DN_NN)


def kernel(x, W_qkv, W_o, W_iq, W_ik):
    b, s, dm = x.shape
    x2 = x[0]
    W = W_qkv.reshape(dm, 3, N_HEADS, D_K)
    perm = jnp.concatenate([jnp.arange(0, D_K, 2), jnp.arange(1, D_K, 2)])
    Wq = W[:, 0][:, :, perm].reshape(dm, D_MODEL)
    Wk = W[:, 1][:, :, perm].reshape(dm, D_MODEL)
    Wv = W[:, 2].reshape(dm, D_MODEL)

    pos = jnp.arange(s, dtype=jnp.float32)
    freqs = 1.0 / (10000.0 ** (jnp.arange(0, D_K, 2, dtype=jnp.float32) / D_K))
    ang = pos[:, None] * freqs[None, :]
    cosv = jnp.cos(ang)
    sinv = jnp.sin(ang)
    cos2 = jnp.concatenate([cosv, cosv], axis=1)
    sin2 = jnp.concatenate([-sinv, sinv], axis=1)

    blk = lambda shape: pl.BlockSpec(shape, lambda i: (i, 0))
    cst = lambda shape: pl.BlockSpec(shape, lambda i: (0, 0))
    f32 = jnp.float32

    q, k = pl.pallas_call(
        _qk_kernel,
        grid=(N_BLK,),
        in_specs=[blk((BQ, D_MODEL)), cst((D_MODEL, D_MODEL)),
                  cst((D_MODEL, D_MODEL)), blk((BQ, D_K)), blk((BQ, D_K))],
        out_specs=[blk((BQ, D_MODEL)), blk((BQ, D_MODEL))],
        out_shape=[jax.ShapeDtypeStruct((S, D_MODEL), f32)] * 2,
    )(x2, Wq, Wk, cos2, sin2)

    v, iq, ik = pl.pallas_call(
        _v_idx_kernel,
        grid=(N_BLK,),
        in_specs=[blk((BQ, D_MODEL)), cst((D_MODEL, D_MODEL)),
                  cst((D_MODEL, IDF)), cst((D_MODEL, IDF))],
        out_specs=[blk((BQ, D_MODEL)), blk((BQ, IDF)), blk((BQ, IDF))],
        out_shape=[jax.ShapeDtypeStruct((S, D_MODEL), f32),
                   jax.ShapeDtypeStruct((S, IDF), f32),
                   jax.ShapeDtypeStruct((S, IDF), f32)],
    )(x2, Wv, W_iq, W_ik)

    bias = pl.pallas_call(
        _select_kernel,
        grid=(N_BLK,),
        in_specs=[blk((BQ, IDF)), cst((S, IDF))],
        out_specs=blk((BQ, S)),
        out_shape=jax.ShapeDtypeStruct((S, S), f32),
    )(iq, ik)

    attnout = pl.pallas_call(
        _attn_kernel,
        grid=(N_BLK,),
        in_specs=[blk((BQ, S)), blk((BQ, D_MODEL)), cst((S, D_MODEL)),
                  cst((S, D_MODEL))],
        out_specs=blk((BQ, D_MODEL)),
        out_shape=jax.ShapeDtypeStruct((S, D_MODEL), f32),
    )(bias, q, k, v)

    out = pl.pallas_call(
        _outproj_kernel,
        grid=(N_BLK,),
        in_specs=[blk((BQ, D_MODEL)), cst((D_MODEL, D_MODEL))],
        out_specs=blk((BQ, D_MODEL)),
        out_shape=jax.ShapeDtypeStruct((S, D_MODEL), f32),
    )(attnout, W_o)

    return out[None, :, :]
